# baseline (speedup 1.0000x reference)
import functools

import jax
import jax.numpy as jnp
from jax.experimental import pallas as pl

MAX_INT = 15.0
BLOCK_ROWS = 16384
CHUNK_ROWS = 128


def _fq_kernel(x_ref, o_ref):
    def body(i, carry):
        xc = x_ref[pl.ds(i * CHUNK_ROWS, CHUNK_ROWS), :]
        mn = jnp.min(xc, axis=-1, keepdims=True)
        mx = jnp.max(xc, axis=-1, keepdims=True)
        scale = jnp.maximum((mx - mn) * (1.0 / MAX_INT), 1e-05)
        inv = 1.0 / scale
        q = jnp.round((xc - mn) * inv)
        o_ref[pl.ds(i * CHUNK_ROWS, CHUNK_ROWS), :] = q * scale + mn
        return carry

    jax.lax.fori_loop(0, BLOCK_ROWS // CHUNK_ROWS, body, 0)


def kernel(tensor):
    bs, num_heads, seqlen, head_dim = tensor.shape
    rows = bs * num_heads * seqlen
    x = tensor.reshape(rows, head_dim)
    out = pl.pallas_call(
        _fq_kernel,
        out_shape=jax.ShapeDtypeStruct((rows, head_dim), tensor.dtype),
        grid=(rows // BLOCK_ROWS,),
        in_specs=[pl.BlockSpec((BLOCK_ROWS, head_dim), lambda i: (i, 0))],
        out_specs=pl.BlockSpec((BLOCK_ROWS, head_dim), lambda i: (i, 0)),
    )(x)
    return out.reshape(bs, num_heads, seqlen, head_dim)


# speedup vs baseline: 2.7240x; 2.7240x over previous
import functools

import jax
import jax.numpy as jnp
from jax.experimental import pallas as pl

MAX_INT = 15.0
BLOCK_ROWS = 16384
CHUNK_ROWS = 128


def _fq_kernel(x_ref, o_ref):
    for i in range(BLOCK_ROWS // CHUNK_ROWS):
        xc = x_ref[i * CHUNK_ROWS:(i + 1) * CHUNK_ROWS, :]
        mn = jnp.min(xc, axis=-1, keepdims=True)
        mx = jnp.max(xc, axis=-1, keepdims=True)
        scale = jnp.maximum((mx - mn) * (1.0 / MAX_INT), 1e-05)
        inv = 1.0 / scale
        q = jnp.round((xc - mn) * inv)
        o_ref[i * CHUNK_ROWS:(i + 1) * CHUNK_ROWS, :] = q * scale + mn


def kernel(tensor):
    bs, num_heads, seqlen, head_dim = tensor.shape
    rows = bs * num_heads * seqlen
    x = tensor.reshape(rows, head_dim)
    out = pl.pallas_call(
        _fq_kernel,
        out_shape=jax.ShapeDtypeStruct((rows, head_dim), tensor.dtype),
        grid=(rows // BLOCK_ROWS,),
        in_specs=[pl.BlockSpec((BLOCK_ROWS, head_dim), lambda i: (i, 0))],
        out_specs=pl.BlockSpec((BLOCK_ROWS, head_dim), lambda i: (i, 0)),
    )(x)
    return out.reshape(bs, num_heads, seqlen, head_dim)
